# baseline (speedup 1.0000x reference)
import jax
import jax.numpy as jnp
from jax import lax
from jax.experimental import pallas as pl
from jax.experimental.pallas import tpu as pltpu
from jax.experimental.pallas import tpu_sc as plsc

MAXLEN = 200
EMBED = 128
BATCH = 1024
NW = 32
BPW = BATCH // NW
TG = 4
NCH = MAXLEN // TG
ROWS = TG * BPW
LANES = 16
NG = 3
NS = 2
STEP = 6
MAIN = NCH - 2


def _body(x_hbm, tok_hbm, pos_hbm, out_hbm, pos_v, idx_v, gbufs, sbufs,
          sgs, sos):
    wid = lax.axis_index("s") * 2 + lax.axis_index("c")
    base = wid * BPW
    pltpu.sync_copy(pos_hbm, pos_v)
    pltpu.sync_copy(x_hbm.at[wid], idx_v)

    def gather(k, g):
        pltpu.async_copy(tok_hbm.at[idx_v.at[k]], gbufs[g], sgs[g])

    def wait_gather(g):
        pltpu.make_async_copy(
            tok_hbm.at[pl.ds(0, ROWS)], gbufs[g], sgs[g]
        ).wait()

    def wait_store(s):
        pltpu.make_async_copy(
            sbufs[s], out_hbm.at[pl.ds(0, BPW), pl.ds(0, TG)], sos[s]
        ).wait()

    def add_and_store(k, g, s):
        slices = [pl.ds(c * LANES, LANES) for c in range(EMBED // LANES)]
        pv = [[pos_v[TG * k + tt, sl] for sl in slices] for tt in range(TG)]

        @pl.loop(0, BPW)
        def _j(j):
            for tt in range(TG):
                for c, sl in enumerate(slices):
                    sbufs[s][j, tt, sl] = gbufs[g][tt * BPW + j, sl] + pv[tt][c]

        pltpu.async_copy(
            sbufs[s], out_hbm.at[pl.ds(base, BPW), pl.ds(TG * k, TG)], sos[s]
        )

    gather(0, 0)
    gather(1, 1)

    @pl.loop(0, MAIN // STEP)
    def _p(p):
        for bb in range(STEP):
            k = STEP * p + bb
            g = bb % NG
            s = bb % NS
            if bb < 2:
                @pl.when(p > 0)
                def _w():
                    wait_store(s)
            else:
                wait_store(s)
            gather(k + 2, (bb + 2) % NG)
            wait_gather(g)
            add_and_store(k, g, s)

    for k in (MAIN, MAIN + 1):
        wait_store(k % NS)
        wait_gather(k % NG)
        add_and_store(k, k % NG, k % NS)
    for s in range(NS):
        wait_store(s)


def _kernel_body(x_hbm, tok_hbm, pos_hbm, out_hbm, pos_v, idx_v,
                 g0, g1, g2, s0, s1, sg0, sg1, sg2, so0, so1):
    _body(x_hbm, tok_hbm, pos_hbm, out_hbm, pos_v, idx_v,
          (g0, g1, g2), (s0, s1), (sg0, sg1, sg2), (so0, so1))


def kernel(x, token_table, pos_table):
    xt = (
        x.reshape(NW, BPW, NCH, TG)
        .transpose(0, 2, 3, 1)
        .reshape(NW, NCH, ROWS)
        .astype(jnp.int32)
    )
    mesh = plsc.VectorSubcoreMesh(core_axis_name="c", subcore_axis_name="s")
    f = pl.kernel(
        _kernel_body,
        out_type=jax.ShapeDtypeStruct((BATCH, MAXLEN, EMBED), jnp.float32),
        mesh=mesh,
        scratch_types=[
            pltpu.VMEM((MAXLEN, EMBED), jnp.float32),
            pltpu.VMEM((NCH, ROWS), jnp.int32),
            pltpu.VMEM((ROWS, EMBED), jnp.float32),
            pltpu.VMEM((ROWS, EMBED), jnp.float32),
            pltpu.VMEM((ROWS, EMBED), jnp.float32),
            pltpu.VMEM((BPW, TG, EMBED), jnp.float32),
            pltpu.VMEM((BPW, TG, EMBED), jnp.float32),
            pltpu.SemaphoreType.DMA,
            pltpu.SemaphoreType.DMA,
            pltpu.SemaphoreType.DMA,
            pltpu.SemaphoreType.DMA,
            pltpu.SemaphoreType.DMA,
        ],
    )
    return f(xt, token_table, pos_table)


# speedup vs baseline: 2.6345x; 2.6345x over previous
import jax
import jax.numpy as jnp
from jax import lax
from jax.experimental import pallas as pl
from jax.experimental.pallas import tpu as pltpu
from jax.experimental.pallas import tpu_sc as plsc

MAXLEN = 200
EMBED = 128
BATCH = 1024
NW = 32
BPW = BATCH // NW
TG = 4
NCH = MAXLEN // TG
ROWS = TG * BPW
LANES = 16
NG = 3
NS = 2
STEP = 6
MAIN = NCH - 2


def _body(x_hbm, tok_hbm, pos_hbm, out_hbm, pos_v, idx_v, gbufs, sbufs,
          sgs, sos):
    wid = lax.axis_index("s") * 2 + lax.axis_index("c")
    base = wid * BPW
    pltpu.sync_copy(pos_hbm, pos_v)
    pltpu.sync_copy(x_hbm.at[wid], idx_v)

    def gather(k, g):
        pltpu.async_copy(tok_hbm.at[idx_v.at[k]], gbufs[g], sgs[g])

    def wait_gather(g):
        pltpu.make_async_copy(
            tok_hbm.at[pl.ds(0, ROWS)], gbufs[g], sgs[g]
        ).wait()

    def wait_store(s):
        pltpu.make_async_copy(
            sbufs[s], out_hbm.at[pl.ds(0, BPW), pl.ds(0, TG)], sos[s]
        ).wait()

    def add_and_store(k, g, s):
        slices = [pl.ds(c * LANES, LANES) for c in range(EMBED // LANES)]
        pv = [[pos_v[TG * k + tt, sl] for sl in slices] for tt in range(TG)]

        @plsc.parallel_loop(0, BPW)
        def _j(j):
            tv = [
                [gbufs[g][tt * BPW + j, sl] for sl in slices]
                for tt in range(TG)
            ]
            for tt in range(TG):
                for c, sl in enumerate(slices):
                    sbufs[s][j, tt, sl] = tv[tt][c] + pv[tt][c]

        pltpu.async_copy(
            sbufs[s], out_hbm.at[pl.ds(base, BPW), pl.ds(TG * k, TG)], sos[s]
        )

    gather(0, 0)
    gather(1, 1)

    @pl.loop(0, MAIN // STEP)
    def _p(p):
        for bb in range(STEP):
            k = STEP * p + bb
            g = bb % NG
            s = bb % NS
            if bb < 2:
                @pl.when(p > 0)
                def _w():
                    wait_store(s)
            else:
                wait_store(s)
            gather(k + 2, (bb + 2) % NG)
            wait_gather(g)
            add_and_store(k, g, s)

    for k in (MAIN, MAIN + 1):
        wait_store(k % NS)
        wait_gather(k % NG)
        add_and_store(k, k % NG, k % NS)
    for s in range(NS):
        wait_store(s)


def _kernel_body(x_hbm, tok_hbm, pos_hbm, out_hbm, pos_v, idx_v,
                 g0, g1, g2, s0, s1, sg0, sg1, sg2, so0, so1):
    _body(x_hbm, tok_hbm, pos_hbm, out_hbm, pos_v, idx_v,
          (g0, g1, g2), (s0, s1), (sg0, sg1, sg2), (so0, so1))


def kernel(x, token_table, pos_table):
    xt = (
        x.reshape(NW, BPW, NCH, TG)
        .transpose(0, 2, 3, 1)
        .reshape(NW, NCH, ROWS)
        .astype(jnp.int32)
    )
    mesh = plsc.VectorSubcoreMesh(core_axis_name="c", subcore_axis_name="s")
    f = pl.kernel(
        _kernel_body,
        out_type=jax.ShapeDtypeStruct((BATCH, MAXLEN, EMBED), jnp.float32),
        mesh=mesh,
        scratch_types=[
            pltpu.VMEM((MAXLEN, EMBED), jnp.float32),
            pltpu.VMEM((NCH, ROWS), jnp.int32),
            pltpu.VMEM((ROWS, EMBED), jnp.float32),
            pltpu.VMEM((ROWS, EMBED), jnp.float32),
            pltpu.VMEM((ROWS, EMBED), jnp.float32),
            pltpu.VMEM((BPW, TG, EMBED), jnp.float32),
            pltpu.VMEM((BPW, TG, EMBED), jnp.float32),
            pltpu.SemaphoreType.DMA,
            pltpu.SemaphoreType.DMA,
            pltpu.SemaphoreType.DMA,
            pltpu.SemaphoreType.DMA,
            pltpu.SemaphoreType.DMA,
        ],
    )
    return f(xt, token_table, pos_table)


# speedup vs baseline: 2.6621x; 1.0105x over previous
import jax
import jax.numpy as jnp
from jax import lax
from jax.experimental import pallas as pl
from jax.experimental.pallas import tpu as pltpu
from jax.experimental.pallas import tpu_sc as plsc

MAXLEN = 200
EMBED = 128
BATCH = 1024
NW = 32
BPW = BATCH // NW
TG = 4
NCH = MAXLEN // TG
ROWS = TG * BPW
LANES = 16
NBUF = 4
MAIN = NCH - 2


def _body(x_hbm, tok_hbm, pos_hbm, out_hbm, pos_v, idx_v, bufs, sgs, sos):
    wid = lax.axis_index("s") * 2 + lax.axis_index("c")
    base = wid * BPW
    pltpu.sync_copy(pos_hbm, pos_v)
    pltpu.sync_copy(x_hbm.at[wid], idx_v)

    def gather(k, b):
        pltpu.async_copy(tok_hbm.at[idx_v.at[k]], bufs[b], sgs[b])

    def wait_gather(b):
        pltpu.make_async_copy(tok_hbm.at[pl.ds(0, ROWS)], bufs[b], sgs[b]).wait()

    def wait_store(b):
        pltpu.make_async_copy(
            bufs[b], out_hbm.at[0, pl.ds(0, ROWS)], sos[b]
        ).wait()

    def add_and_store(k, b):
        slices = [pl.ds(c * LANES, LANES) for c in range(EMBED // LANES)]
        pv = [
            [pos_v[TG * k + tt, sl] for sl in slices] for tt in range(TG)
        ]

        @pl.loop(0, BPW)
        def _j(j):
            for tt in range(TG):
                r = tt * BPW + j
                for c, sl in enumerate(slices):
                    bufs[b][r, sl] = bufs[b][r, sl] + pv[tt][c]

        for tt in range(TG):
            pltpu.async_copy(
                bufs[b].at[pl.ds(tt * BPW, BPW)],
                out_hbm.at[pl.ds(base, BPW), TG * k + tt],
                sos[b],
            )

    gather(0, 0)
    gather(1, 1)

    @pl.loop(0, MAIN // NBUF)
    def _p(p):
        for bb in range(NBUF):
            k = NBUF * p + bb
            nb = (bb + 2) % NBUF
            if bb < 2:
                @pl.when(p > 0)
                def _w():
                    wait_store(nb)
            else:
                wait_store(nb)
            gather(k + 2, nb)
            wait_gather(bb)
            add_and_store(k, bb)

    for k in (MAIN, MAIN + 1):
        b = k % NBUF
        wait_gather(b)
        add_and_store(k, b)
    for b in range(NBUF):
        wait_store(b)


def _kernel_body(x_hbm, tok_hbm, pos_hbm, out_hbm, pos_v, idx_v,
                 buf0, buf1, buf2, buf3, sg0, sg1, sg2, sg3,
                 so0, so1, so2, so3):
    _body(x_hbm, tok_hbm, pos_hbm, out_hbm, pos_v, idx_v,
          (buf0, buf1, buf2, buf3), (sg0, sg1, sg2, sg3),
          (so0, so1, so2, so3))


def kernel(x, token_table, pos_table):
    xt = (
        x.reshape(NW, BPW, NCH, TG)
        .transpose(0, 2, 3, 1)
        .reshape(NW, NCH, ROWS)
        .astype(jnp.int32)
    )
    mesh = plsc.VectorSubcoreMesh(core_axis_name="c", subcore_axis_name="s")
    f = pl.kernel(
        _kernel_body,
        out_type=jax.ShapeDtypeStruct((BATCH, MAXLEN, EMBED), jnp.float32),
        mesh=mesh,
        scratch_types=[
            pltpu.VMEM((MAXLEN, EMBED), jnp.float32),
            pltpu.VMEM((NCH, ROWS), jnp.int32),
            pltpu.VMEM((ROWS, EMBED), jnp.float32),
            pltpu.VMEM((ROWS, EMBED), jnp.float32),
            pltpu.VMEM((ROWS, EMBED), jnp.float32),
            pltpu.VMEM((ROWS, EMBED), jnp.float32),
            pltpu.SemaphoreType.DMA,
            pltpu.SemaphoreType.DMA,
            pltpu.SemaphoreType.DMA,
            pltpu.SemaphoreType.DMA,
            pltpu.SemaphoreType.DMA,
            pltpu.SemaphoreType.DMA,
            pltpu.SemaphoreType.DMA,
            pltpu.SemaphoreType.DMA,
        ],
    )
    return f(xt, token_table, pos_table)
